# Initial kernel scaffold; baseline (speedup 1.0000x reference)
import jax
import jax.numpy as jnp
from jax.experimental import pallas as pl


def kernel(z, mask, nodes):
    raise NotImplementedError("write your pallas kernel here")


# speedup vs baseline: 4.3212x; 4.3212x over previous
import functools

import jax
import jax.numpy as jnp
from jax.experimental import pallas as pl

GRID = (32, 32)
LATENT_DIM = 64
ALPHA = 1.0
TIME_DECAY = 0.99
MAX_SEQ_LEN = 512

_N = GRID[0] * GRID[1]


def _som_block(z_ref, tw_ref, mask_ref, nodes_ref,
               som_ref, q_ref, bmu_ref, k_ref):
    z = z_ref[...]
    m = mask_ref[...]
    wz = z * tw_ref[...] * m
    nodes = nodes_ref[...]

    g = jax.lax.dot_general(wz, nodes, (((1,), (1,)), ((), ())),
                            precision=jax.lax.Precision.HIGHEST,
                            preferred_element_type=jnp.float32)
    zsq = jnp.sum(wz * wz, axis=1, keepdims=True)
    nsq = jnp.sum(nodes * nodes, axis=1)[None, :]
    d2 = jnp.maximum(zsq + nsq - 2.0 * g, 0.0)
    dist = jnp.sqrt(d2)

    q = 1.0 / (1.0 + dist / ALPHA)
    qs = jnp.sum(q, axis=1, keepdims=True)
    q_ref[...] = q / jnp.maximum(qs, 1e-12)

    dmin = jnp.min(dist, axis=1, keepdims=True)
    idx = jax.lax.broadcasted_iota(jnp.int32, dist.shape, 1)
    bmu = jnp.min(jnp.where(dist == dmin, idx, jnp.int32(2 ** 30)), axis=1)
    bmu_ref[...] = bmu[:, None]
    k_ref[...] = jnp.concatenate(
        [(bmu // GRID[1])[:, None], (bmu % GRID[1])[:, None]], axis=1)

    onehot = (idx == bmu[:, None]).astype(jnp.float32)
    gathered = jax.lax.dot_general(onehot, nodes, (((1,), (0,)), ((), ())),
                                   precision=jax.lax.Precision.HIGHEST,
                                   preferred_element_type=jnp.float32)
    som_ref[...] = z + 0.1 * (gathered - z) * m


@functools.partial(jax.jit, static_argnames=())
def kernel(z, mask, nodes):
    B, T, D = z.shape
    R = 256
    rows = B * T

    t_idx = jnp.arange(MAX_SEQ_LEN, dtype=jnp.float32)
    tw = (TIME_DECAY ** (MAX_SEQ_LEN - t_idx - 1.0)).astype(jnp.float32)
    tw = tw[MAX_SEQ_LEN - T:]
    tw_full = jnp.tile(tw, (B,)).reshape(rows, 1)

    z_flat = z.reshape(rows, D)
    mask_flat = mask.reshape(rows, 1)
    nodes_flat = nodes.reshape(_N, D)

    grid = (rows // R,)
    som, q, bmu, k = pl.pallas_call(
        _som_block,
        grid=grid,
        in_specs=[
            pl.BlockSpec((R, D), lambda i: (i, 0)),
            pl.BlockSpec((R, 1), lambda i: (i, 0)),
            pl.BlockSpec((R, 1), lambda i: (i, 0)),
            pl.BlockSpec((_N, D), lambda i: (0, 0)),
        ],
        out_specs=[
            pl.BlockSpec((R, D), lambda i: (i, 0)),
            pl.BlockSpec((R, _N), lambda i: (i, 0)),
            pl.BlockSpec((R, 1), lambda i: (i, 0)),
            pl.BlockSpec((R, 2), lambda i: (i, 0)),
        ],
        out_shape=[
            jax.ShapeDtypeStruct((rows, D), jnp.float32),
            jax.ShapeDtypeStruct((rows, _N), jnp.float32),
            jax.ShapeDtypeStruct((rows, 1), jnp.int32),
            jax.ShapeDtypeStruct((rows, 2), jnp.int32),
        ],
    )(z_flat, tw_full, mask_flat, nodes_flat)

    som_z = som.reshape(B, T, D)
    bmu_b = bmu.reshape(B, T)
    k_out = k.reshape(B, T, 2)
    return (som_z, q, bmu_b, k_out)


# speedup vs baseline: 6.6722x; 1.5441x over previous
import functools

import jax
import jax.numpy as jnp
from jax.experimental import pallas as pl

GRID = (32, 32)
LATENT_DIM = 64
ALPHA = 1.0
TIME_DECAY = 0.99
MAX_SEQ_LEN = 512

_N = GRID[0] * GRID[1]


def _som_block(z_ref, tw_ref, mask_ref, nodes_ref,
               som_ref, q_ref, bmu_ref, k_ref):
    z = z_ref[...]
    m = mask_ref[...]
    wz = z * tw_ref[...] * m
    nodes = nodes_ref[...]

    g = jax.lax.dot_general(wz, nodes, (((1,), (1,)), ((), ())),
                            precision=jax.lax.Precision.HIGHEST,
                            preferred_element_type=jnp.float32)
    zsq = jnp.sum(wz * wz, axis=1, keepdims=True)
    nsq = jnp.sum(nodes * nodes, axis=1)[None, :]
    d2 = jnp.maximum(zsq + nsq - 2.0 * g, 0.0)
    dist = jnp.sqrt(d2)

    q = 1.0 / (1.0 + dist / ALPHA)
    qs = jnp.sum(q, axis=1, keepdims=True)
    q_ref[...] = q / jnp.maximum(qs, 1e-12)

    bmu = jnp.argmin(dist, axis=1).astype(jnp.int32)
    bmu_ref[...] = bmu[:, None]
    k_ref[...] = jnp.concatenate(
        [(bmu // GRID[1])[:, None], (bmu % GRID[1])[:, None]], axis=1)

    idx = jax.lax.broadcasted_iota(jnp.int32, dist.shape, 1)
    onehot = (idx == bmu[:, None]).astype(jnp.float32)
    gathered = jax.lax.dot_general(onehot, nodes, (((1,), (0,)), ((), ())),
                                   preferred_element_type=jnp.float32)
    som_ref[...] = z + 0.1 * (gathered - z) * m


@functools.partial(jax.jit, static_argnames=())
def kernel(z, mask, nodes):
    B, T, D = z.shape
    R = 256
    rows = B * T

    t_idx = jnp.arange(MAX_SEQ_LEN, dtype=jnp.float32)
    tw = (TIME_DECAY ** (MAX_SEQ_LEN - t_idx - 1.0)).astype(jnp.float32)
    tw = tw[MAX_SEQ_LEN - T:]
    tw_full = jnp.tile(tw, (B,)).reshape(rows, 1)

    z_flat = z.reshape(rows, D)
    mask_flat = mask.reshape(rows, 1)
    nodes_flat = nodes.reshape(_N, D)

    grid = (rows // R,)
    som, q, bmu, k = pl.pallas_call(
        _som_block,
        grid=grid,
        in_specs=[
            pl.BlockSpec((R, D), lambda i: (i, 0)),
            pl.BlockSpec((R, 1), lambda i: (i, 0)),
            pl.BlockSpec((R, 1), lambda i: (i, 0)),
            pl.BlockSpec((_N, D), lambda i: (0, 0)),
        ],
        out_specs=[
            pl.BlockSpec((R, D), lambda i: (i, 0)),
            pl.BlockSpec((R, _N), lambda i: (i, 0)),
            pl.BlockSpec((R, 1), lambda i: (i, 0)),
            pl.BlockSpec((R, 2), lambda i: (i, 0)),
        ],
        out_shape=[
            jax.ShapeDtypeStruct((rows, D), jnp.float32),
            jax.ShapeDtypeStruct((rows, _N), jnp.float32),
            jax.ShapeDtypeStruct((rows, 1), jnp.int32),
            jax.ShapeDtypeStruct((rows, 2), jnp.int32),
        ],
    )(z_flat, tw_full, mask_flat, nodes_flat)

    som_z = som.reshape(B, T, D)
    bmu_b = bmu.reshape(B, T)
    k_out = k.reshape(B, T, 2)
    return (som_z, q, bmu_b, k_out)


# speedup vs baseline: 6.8595x; 1.0281x over previous
import functools

import jax
import jax.numpy as jnp
from jax.experimental import pallas as pl

GRID = (32, 32)
LATENT_DIM = 64
ALPHA = 1.0
TIME_DECAY = 0.99
MAX_SEQ_LEN = 512

_N = GRID[0] * GRID[1]


def _som_block(z_ref, tw_ref, mask_ref, nodes_ref,
               som_ref, q_ref, bmu_ref, k_ref):
    z = z_ref[...]
    m = mask_ref[...]
    wz = z * tw_ref[...] * m
    nodes = nodes_ref[...]

    g = jax.lax.dot_general(wz, nodes, (((1,), (1,)), ((), ())),
                            precision=jax.lax.Precision.HIGHEST,
                            preferred_element_type=jnp.float32)
    zsq = jnp.sum(wz * wz, axis=1, keepdims=True)
    nsq = jnp.sum(nodes * nodes, axis=1)[None, :]
    d2 = jnp.maximum(zsq + nsq - 2.0 * g, 0.0)
    dist = jnp.sqrt(d2)

    q = 1.0 / (1.0 + dist / ALPHA)
    qs = jnp.sum(q, axis=1, keepdims=True)
    q_ref[...] = q / jnp.maximum(qs, 1e-12)

    bmu = jnp.argmin(dist, axis=1).astype(jnp.int32)
    bmu_ref[...] = bmu[:, None]
    k_ref[...] = jnp.concatenate(
        [(bmu // GRID[1])[:, None], (bmu % GRID[1])[:, None]], axis=1)

    idx = jax.lax.broadcasted_iota(jnp.int32, dist.shape, 1)
    onehot = (idx == bmu[:, None]).astype(jnp.float32)
    gathered = jax.lax.dot_general(onehot, nodes, (((1,), (0,)), ((), ())),
                                   preferred_element_type=jnp.float32)
    som_ref[...] = z + 0.1 * (gathered - z) * m


@functools.partial(jax.jit, static_argnames=())
def kernel(z, mask, nodes):
    B, T, D = z.shape
    R = 512
    rows = B * T

    t_idx = jnp.arange(MAX_SEQ_LEN, dtype=jnp.float32)
    tw = (TIME_DECAY ** (MAX_SEQ_LEN - t_idx - 1.0)).astype(jnp.float32)
    tw = tw[MAX_SEQ_LEN - T:]
    tw_full = jnp.tile(tw, (B,)).reshape(rows, 1)

    z_flat = z.reshape(rows, D)
    mask_flat = mask.reshape(rows, 1)
    nodes_flat = nodes.reshape(_N, D)

    grid = (rows // R,)
    som, q, bmu, k = pl.pallas_call(
        _som_block,
        grid=grid,
        in_specs=[
            pl.BlockSpec((R, D), lambda i: (i, 0)),
            pl.BlockSpec((R, 1), lambda i: (i, 0)),
            pl.BlockSpec((R, 1), lambda i: (i, 0)),
            pl.BlockSpec((_N, D), lambda i: (0, 0)),
        ],
        out_specs=[
            pl.BlockSpec((R, D), lambda i: (i, 0)),
            pl.BlockSpec((R, _N), lambda i: (i, 0)),
            pl.BlockSpec((R, 1), lambda i: (i, 0)),
            pl.BlockSpec((R, 2), lambda i: (i, 0)),
        ],
        out_shape=[
            jax.ShapeDtypeStruct((rows, D), jnp.float32),
            jax.ShapeDtypeStruct((rows, _N), jnp.float32),
            jax.ShapeDtypeStruct((rows, 1), jnp.int32),
            jax.ShapeDtypeStruct((rows, 2), jnp.int32),
        ],
    )(z_flat, tw_full, mask_flat, nodes_flat)

    som_z = som.reshape(B, T, D)
    bmu_b = bmu.reshape(B, T)
    k_out = k.reshape(B, T, 2)
    return (som_z, q, bmu_b, k_out)
